# baseline (speedup 1.0000x reference)
import functools

import jax
import jax.numpy as jnp
from jax import lax
from jax.experimental import pallas as pl
from jax.experimental.pallas import tpu as pltpu
from jax.experimental.pallas import tpu_sc as plsc

_BATCH = 16384
_STRIDE0 = 10000
_STRIDE1 = 100
_NC = 2
_NS = 16
_NW = _NC * _NS
_BPW = _BATCH // _NW
_GCHUNK = 128
_NGATHER = _BPW // _GCHUNK

_DNUMS = lax.GatherDimensionNumbers(
    offset_dims=(), collapsed_slice_dims=(0,), start_index_map=(0,)
)


def _dg(v, i):
    return lax.gather(
        v,
        i.reshape(16, 1),
        _DNUMS,
        (1,),
        mode=lax.GatherScatterMode.PROMISE_IN_BOUNDS,
    )


def _sc_lookup_body(xs_hbm, table_hbm, out_hbm, xs_v, idx_v, out_v, sem):
    wid = lax.axis_index("s") * _NC + lax.axis_index("c")
    base = wid * _BPW

    pltpu.sync_copy(xs_hbm.at[pl.ds(base * 3, _BPW * 3)], xs_v)

    lane = lax.iota(jnp.int32, 16)
    perms = []
    for j in range(3):
        w = lane * 3 + j
        perms.append(
            (
                jnp.clip(w, 0, 15),
                jnp.clip(w - 16, 0, 15),
                jnp.clip(w - 32, 0, 15),
                w < 16,
                w < 32,
            )
        )

    def deint(a, b, d, p):
        ia, ib, ic, in_a, in_ab = p
        return jnp.where(in_a, _dg(a, ia), jnp.where(in_ab, _dg(b, ib), _dg(d, ic)))

    copies = []
    for k in range(_NGATHER):
        for c8 in range(_GCHUNK // 16):
            c = k * (_GCHUNK // 16) + c8
            a = xs_v[pl.ds(c * 48, 16)]
            b = xs_v[pl.ds(c * 48 + 16, 16)]
            d = xs_v[pl.ds(c * 48 + 32, 16)]
            x0 = deint(a, b, d, perms[0])
            x1 = deint(a, b, d, perms[1])
            x2 = deint(a, b, d, perms[2])
            idx_v[k, pl.ds(c8 * 16, 16)] = x0 * _STRIDE0 + x1 * _STRIDE1 + x2
        copies.append(
            pltpu.async_copy(
                table_hbm.at[idx_v.at[k]], out_v.at[pl.ds(k * _GCHUNK, _GCHUNK)], sem
            )
        )
    for cp in copies:
        cp.wait()

    pltpu.sync_copy(out_v, out_hbm.at[pl.ds(base, _BPW)])


@functools.partial(
    pl.kernel,
    out_type=jax.ShapeDtypeStruct((_BATCH,), jnp.float32),
    mesh=plsc.VectorSubcoreMesh(
        core_axis_name="c", subcore_axis_name="s", num_cores=_NC, num_subcores=_NS
    ),
    scratch_types=[
        pltpu.VMEM((_BPW * 3,), jnp.int32),
        pltpu.VMEM((_NGATHER, _GCHUNK), jnp.int32),
        pltpu.VMEM((_BPW,), jnp.float32),
        pltpu.SemaphoreType.DMA,
    ],
)
def _sc_lookup(xs_hbm, table_hbm, out_hbm, xs_v, idx_v, out_v, sem):
    _sc_lookup_body(xs_hbm, table_hbm, out_hbm, xs_v, idx_v, out_v, sem)


def kernel(xs, embed_weight):
    xs_flat = xs.reshape(-1)
    table = embed_weight.reshape(-1)
    return _sc_lookup(xs_flat, table)


# speedup vs baseline: 1.1671x; 1.1671x over previous
import functools

import jax
import jax.numpy as jnp
from jax import lax
from jax.experimental import pallas as pl
from jax.experimental.pallas import tpu as pltpu
from jax.experimental.pallas import tpu_sc as plsc

_BATCH = 16384
_STRIDE0 = 10000
_STRIDE1 = 100
_NC = 2
_NS = 16
_NW = _NC * _NS
_BPW = _BATCH // _NW
_GCHUNK = 128
_NGATHER = _BPW // _GCHUNK


def _sc_lookup_body(xs_hbm, table_hbm, out_hbm, xs_v, idx_v, out_v, sem):
    wid = lax.axis_index("s") * _NC + lax.axis_index("c")
    base = wid * _BPW

    pltpu.sync_copy(xs_hbm.at[pl.ds(base * 3, _BPW * 3)], xs_v)

    copies = []
    for k in range(_NGATHER):
        for c8 in range(_GCHUNK // 16):
            off = k * _GCHUNK + c8 * 16
            x0 = xs_v[pl.ds(off, 16)]
            x1 = xs_v[pl.ds(_BPW + off, 16)]
            x2 = xs_v[pl.ds(2 * _BPW + off, 16)]
            idx_v[k, pl.ds(c8 * 16, 16)] = x0 * _STRIDE0 + x1 * _STRIDE1 + x2
        copies.append(
            pltpu.async_copy(
                table_hbm.at[idx_v.at[k]], out_v.at[pl.ds(k * _GCHUNK, _GCHUNK)], sem
            )
        )
    for cp in copies:
        cp.wait()

    pltpu.sync_copy(out_v, out_hbm.at[pl.ds(base, _BPW)])


@functools.partial(
    pl.kernel,
    out_type=jax.ShapeDtypeStruct((_BATCH,), jnp.float32),
    mesh=plsc.VectorSubcoreMesh(
        core_axis_name="c", subcore_axis_name="s", num_cores=_NC, num_subcores=_NS
    ),
    scratch_types=[
        pltpu.VMEM((_BPW * 3,), jnp.int32),
        pltpu.VMEM((_NGATHER, _GCHUNK), jnp.int32),
        pltpu.VMEM((_BPW,), jnp.float32),
        pltpu.SemaphoreType.DMA,
    ],
)
def _sc_lookup(xs_hbm, table_hbm, out_hbm, xs_v, idx_v, out_v, sem):
    _sc_lookup_body(xs_hbm, table_hbm, out_hbm, xs_v, idx_v, out_v, sem)


def kernel(xs, embed_weight):
    xs_r = xs.reshape(_NW, _BPW, 3).transpose(0, 2, 1).reshape(-1)
    table = embed_weight.reshape(-1)
    return _sc_lookup(xs_r, table)


# speedup vs baseline: 1.1693x; 1.0018x over previous
import functools

import jax
import jax.numpy as jnp
from jax import lax
from jax.experimental import pallas as pl
from jax.experimental.pallas import tpu as pltpu
from jax.experimental.pallas import tpu_sc as plsc

_BATCH = 16384
_STRIDE0 = 10000
_STRIDE1 = 100
_NC = 2
_NS = 16
_NW = _NC * _NS
_BPW = _BATCH // _NW
_GCHUNK = 128
_NGATHER = _BPW // _GCHUNK
_CWORDS = 3 * _GCHUNK


def _sc_lookup_body(xs_hbm, table_hbm, out_hbm, xs_v, idx_v, out_v, in_sem, g_sem):
    wid = lax.axis_index("s") * _NC + lax.axis_index("c")
    base = wid * _BPW

    in_copies = [
        pltpu.async_copy(
            xs_hbm.at[pl.ds((wid * _NGATHER + k) * _CWORDS, _CWORDS)],
            xs_v.at[k],
            in_sem.at[k],
        )
        for k in range(_NGATHER)
    ]

    gathers = []
    for k in range(_NGATHER):
        in_copies[k].wait()
        for c8 in range(_GCHUNK // 16):
            off = c8 * 16
            x0 = xs_v[k, pl.ds(off, 16)]
            x1 = xs_v[k, pl.ds(_GCHUNK + off, 16)]
            x2 = xs_v[k, pl.ds(2 * _GCHUNK + off, 16)]
            idx_v[k, pl.ds(off, 16)] = x0 * _STRIDE0 + x1 * _STRIDE1 + x2
        gathers.append(
            pltpu.async_copy(
                table_hbm.at[idx_v.at[k]], out_v.at[pl.ds(k * _GCHUNK, _GCHUNK)], g_sem
            )
        )
    for g in gathers:
        g.wait()

    pltpu.sync_copy(out_v, out_hbm.at[pl.ds(base, _BPW)])


@functools.partial(
    pl.kernel,
    out_type=jax.ShapeDtypeStruct((_BATCH,), jnp.float32),
    mesh=plsc.VectorSubcoreMesh(
        core_axis_name="c", subcore_axis_name="s", num_cores=_NC, num_subcores=_NS
    ),
    scratch_types=[
        pltpu.VMEM((_NGATHER, _CWORDS), jnp.int32),
        pltpu.VMEM((_NGATHER, _GCHUNK), jnp.int32),
        pltpu.VMEM((_BPW,), jnp.float32),
        pltpu.SemaphoreType.DMA((_NGATHER,)),
        pltpu.SemaphoreType.DMA,
    ],
)
def _sc_lookup(xs_hbm, table_hbm, out_hbm, xs_v, idx_v, out_v, in_sem, g_sem):
    _sc_lookup_body(xs_hbm, table_hbm, out_hbm, xs_v, idx_v, out_v, in_sem, g_sem)


def kernel(xs, embed_weight):
    xs_r = xs.reshape(_NW, _NGATHER, _GCHUNK, 3).transpose(0, 1, 3, 2).reshape(-1)
    table = embed_weight.reshape(-1)
    return _sc_lookup(xs_r, table)
